# Initial kernel scaffold; baseline (speedup 1.0000x reference)
import jax
import jax.numpy as jnp
from jax.experimental import pallas as pl


def kernel(s, v, edge_index, edge_attr, edge_vec_unit, ng_W1, ng_b1, ng_W2, ng_b2, eg_W1, eg_b1, eg_W2, eg_b2, mg_W1, mg_b1, mg_W2, mg_b2, pe_W, pe_b, up_W, up_b, ln_g, ln_b):
    raise NotImplementedError("write your pallas kernel here")


# speedup vs baseline: 3.0168x; 3.0168x over previous
import functools

import jax
import jax.numpy as jnp
from jax import lax
from jax.experimental import pallas as pl
from jax.experimental.pallas import tpu as pltpu
from jax.experimental.pallas import tpu_sc as plsc

N = 10000
E = 320000
D = 128
NC = 2
NS = 16
NW = NC * NS
CH = 128
NCHUNK = E // CH
ROWS_PER_TILE = 632
ROWS_LAST = N - 15 * ROWS_PER_TILE
BE = 512
NBE = E // BE


def _prep_body(s_ref, ngW1d_ref, ngb1_ref, ngW2_ref, egW2_ref, mgW1_ref,
               mgb1_ref, ngb2_ref, egb2_ref,
               h_ref, ci_ref, cj_ref, ce_ref, zb_ref):
    f32 = jnp.float32
    h_ref[...] = (jnp.dot(s_ref[...], ngW1d_ref[...], preferred_element_type=f32)
                  + ngb1_ref[...])
    mgW1 = mgW1_ref[...]
    A = mgW1[0:D, :]
    B = mgW1[D:2 * D, :]
    C = mgW1[2 * D:3 * D, :]
    ci_ref[...] = jnp.dot(ngW2_ref[...], A, preferred_element_type=f32)
    cj_ref[...] = jnp.dot(ngW2_ref[...], B, preferred_element_type=f32)
    ce_ref[...] = jnp.dot(egW2_ref[...], C, preferred_element_type=f32)
    zb_ref[...] = (jnp.dot(ngb2_ref[...], A + B, preferred_element_type=f32)
                   + jnp.dot(egb2_ref[...], C, preferred_element_type=f32)
                   + mgb1_ref[...])


def _edge_sc_body(h_hbm, vflat_hbm, ei_hbm, evut_hbm,
                  hi_hbm, hj_hbm, rij_hbm, rji_hbm, cm2_hbm,
                  vtab, rowb, colb, evub, hib, hjb, rijb, rjib, cm2b,
                  sem1, sem2):
    cid = lax.axis_index("c")
    sid = lax.axis_index("s")
    w = sid * NC + cid
    pltpu.sync_copy(vflat_hbm, vtab)
    nmy = (NCHUNK - 1 - w) // NW + 1

    def step(k, carry):
        c = w + k * NW
        off = c * CH
        pltpu.sync_copy(ei_hbm.at[0, pl.ds(off, CH)], rowb)
        pltpu.sync_copy(ei_hbm.at[1, pl.ds(off, CH)], colb)
        pltpu.sync_copy(evut_hbm.at[:, pl.ds(off, CH)], evub)
        cp1 = pltpu.async_copy(h_hbm.at[rowb], hib, sem1)
        cp2 = pltpu.async_copy(h_hbm.at[colb], hjb, sem2)
        for g in range(CH // 16):
            sl = pl.ds(g * 16, 16)
            r16 = rowb[sl]
            c16 = colb[sl]
            vix = plsc.load_gather(vtab, [r16])
            viy = plsc.load_gather(vtab, [r16 + N])
            viz = plsc.load_gather(vtab, [r16 + 2 * N])
            vjx = plsc.load_gather(vtab, [c16])
            vjy = plsc.load_gather(vtab, [c16 + N])
            vjz = plsc.load_gather(vtab, [c16 + 2 * N])
            ex = evub[0, sl]
            ey = evub[1, sl]
            ez = evub[2, sl]
            rijb[sl] = 1.0 - (vix * ex + viy * ey + viz * ez)
            rjib[sl] = 1.0 + (vjx * ex + vjy * ey + vjz * ez)
            cx = viy * vjz - viz * vjy
            cy = viz * vjx - vix * vjz
            cz = vix * vjy - viy * vjx
            cm2b[sl] = cx * cx + cy * cy + cz * cz
        pltpu.sync_copy(rijb, rij_hbm.at[pl.ds(off, CH)])
        pltpu.sync_copy(rjib, rji_hbm.at[pl.ds(off, CH)])
        pltpu.sync_copy(cm2b, cm2_hbm.at[pl.ds(off, CH)])
        cp1.wait()
        cp2.wait()
        pltpu.sync_copy(hib, hi_hbm.at[pl.ds(off, CH), :])
        pltpu.sync_copy(hjb, hj_hbm.at[pl.ds(off, CH), :])
        return carry

    lax.fori_loop(0, nmy, step, 0)


def _mlp_body(hi_ref, hj_ref, ea_ref, rij_ref, rji_ref, cm2_ref,
              wng_ref, weg_ref, egW1d_ref, egb1_ref,
              ci_ref, cj_ref, ce_ref, zb_ref, mgW2_ref, mgb2_ref,
              peW_ref, peb_ref,
              msg_ref, coeff_ref):
    f32 = jnp.float32
    rij = rij_ref[...]
    rji = rji_ref[...]
    cm = jnp.sqrt(cm2_ref[...])
    wng = wng_ref[...]
    ai = jax.nn.silu(hi_ref[...] + rij * wng)
    aj = jax.nn.silu(hj_ref[...] + rji * wng)
    t = (jnp.dot(ea_ref[...], egW1d_ref[...], preferred_element_type=f32)
         + cm * weg_ref[...] + egb1_ref[...])
    ae = jax.nn.silu(t)
    z = (jnp.dot(ai, ci_ref[...], preferred_element_type=f32)
         + jnp.dot(aj, cj_ref[...], preferred_element_type=f32)
         + jnp.dot(ae, ce_ref[...], preferred_element_type=f32)
         + zb_ref[...])
    m = jax.nn.silu(z)
    msg = jnp.dot(m, mgW2_ref[...], preferred_element_type=f32) + mgb2_ref[...]
    msg_ref[...] = msg
    coeff_ref[...] = jnp.dot(msg, peW_ref[...], preferred_element_type=f32) + peb_ref[...]


def _scatter_sc_body(msg_hbm, coeff_hbm, evut_hbm, row_hbm, zs_hbm, zv_hbm,
                     spart_hbm, vpart_hbm,
                     sacc, vacc, msgb, rowb, cofb, evub, valb, idxb):
    cid = lax.axis_index("c")
    sid = lax.axis_index("s")
    w = sid * NC + cid
    r0 = sid * ROWS_PER_TILE

    @pl.when(sid < NS - 1)
    def _():
        pltpu.sync_copy(zs_hbm.at[pl.ds(r0, ROWS_PER_TILE), :],
                        sacc.at[pl.ds(r0, ROWS_PER_TILE), :])

    @pl.when(sid == NS - 1)
    def _():
        pltpu.sync_copy(zs_hbm.at[pl.ds(r0, ROWS_LAST), :],
                        sacc.at[pl.ds(r0, ROWS_LAST), :])

    @pl.when(sid == 0)
    def _():
        pltpu.sync_copy(zv_hbm, vacc)

    plsc.subcore_barrier()
    nmy = (NCHUNK - 1 - w) // NW + 1

    def step(k, carry):
        c = w + k * NW
        off = c * CH
        pltpu.sync_copy(msg_hbm.at[pl.ds(off, CH), :], msgb)
        pltpu.sync_copy(row_hbm.at[pl.ds(off, CH)], rowb)
        pltpu.sync_copy(coeff_hbm.at[pl.ds(off, CH)], cofb)
        pltpu.sync_copy(evut_hbm.at[:, pl.ds(off, CH)], evub)
        pltpu.sync_copy(msgb, sacc.at[rowb], add=True)
        for g in range(CH // 16):
            sl = pl.ds(g * 16, 16)
            r16 = rowb[sl]
            cf = cofb[sl]
            idxb[0, sl] = r16
            idxb[1, sl] = r16 + N
            idxb[2, sl] = r16 + 2 * N
            valb[0, sl] = evub[0, sl] * cf
            valb[1, sl] = evub[1, sl] * cf
            valb[2, sl] = evub[2, sl] * cf
        for comp in range(3):
            pltpu.sync_copy(valb.at[comp], vacc.at[idxb.at[comp]], add=True)
        return carry

    lax.fori_loop(0, nmy, step, 0)
    plsc.subcore_barrier()

    @pl.when(sid < NS - 1)
    def _():
        pltpu.sync_copy(sacc.at[pl.ds(r0, ROWS_PER_TILE), :],
                        spart_hbm.at[cid, pl.ds(r0, ROWS_PER_TILE), :])

    @pl.when(sid == NS - 1)
    def _():
        pltpu.sync_copy(sacc.at[pl.ds(r0, ROWS_LAST), :],
                        spart_hbm.at[cid, pl.ds(r0, ROWS_LAST), :])

    @pl.when(sid == 0)
    def _():
        pltpu.sync_copy(vacc, vpart_hbm.at[cid])


def _final_body(s_ref, spart_ref, vt_ref, vpart_ref, upW_ref, upb_ref,
                lng_ref, lnb_ref, snew_ref, vnewt_ref):
    f32 = jnp.float32
    s_out = spart_ref[0] + spart_ref[1]
    q = (jnp.dot(jax.nn.silu(s_out), upW_ref[...], preferred_element_type=f32)
         + upb_ref[...] + s_ref[...])
    mean = jnp.mean(q, axis=-1, keepdims=True)
    var = jnp.mean((q - mean) ** 2, axis=-1, keepdims=True)
    snew_ref[...] = (q - mean) / jnp.sqrt(var + 1e-5) * lng_ref[...] + lnb_ref[...]
    vn = vt_ref[...] + vpart_ref[0] + vpart_ref[1]
    nx = vn[0:1, :]
    ny = vn[1:2, :]
    nz = vn[2:3, :]
    denom = jnp.maximum(jnp.sqrt(nx * nx + ny * ny + nz * nz), 1e-6)
    vnewt_ref[...] = vn / denom


def kernel(s, v, edge_index, edge_attr, edge_vec_unit,
           ng_W1, ng_b1, ng_W2, ng_b2,
           eg_W1, eg_b1, eg_W2, eg_b2,
           mg_W1, mg_b1, mg_W2, mg_b2,
           pe_W, pe_b, up_W, up_b, ln_g, ln_b):
    f32 = jnp.float32

    vt = v.T.reshape(3, N)
    vflat = vt.reshape(3 * N)
    evut = edge_vec_unit.T.reshape(3, E)
    row = edge_index[0]
    ngW1d = ng_W1[:D]
    wng = ng_W1[D:D + 1]
    egW1d = eg_W1[:D]
    weg = eg_W1[D:D + 1]
    zeros_s = jnp.zeros((N, D), f32)
    zeros_v = jnp.zeros((3 * N,), f32)

    h, ci, cj, ce, zb = pl.pallas_call(
        _prep_body,
        out_shape=[
            jax.ShapeDtypeStruct((N, D), f32),
            jax.ShapeDtypeStruct((D, 2 * D), f32),
            jax.ShapeDtypeStruct((D, 2 * D), f32),
            jax.ShapeDtypeStruct((D, 2 * D), f32),
            jax.ShapeDtypeStruct((1, 2 * D), f32),
        ],
    )(s, ngW1d, ng_b1.reshape(1, D), ng_W2, eg_W2, mg_W1,
      mg_b1.reshape(1, 2 * D), ng_b2.reshape(1, D), eg_b2.reshape(1, D))

    mesh = plsc.VectorSubcoreMesh(core_axis_name="c", subcore_axis_name="s",
                                  num_cores=NC, num_subcores=NS)
    sc_params = pltpu.CompilerParams(needs_layout_passes=False)
    edge_sc = pl.kernel(
        _edge_sc_body,
        compiler_params=sc_params,
        out_type=[
            jax.ShapeDtypeStruct((E, D), f32),
            jax.ShapeDtypeStruct((E, D), f32),
            jax.ShapeDtypeStruct((E,), f32),
            jax.ShapeDtypeStruct((E,), f32),
            jax.ShapeDtypeStruct((E,), f32),
        ],
        mesh=mesh,
        scratch_types=[
            pltpu.VMEM((3 * N,), f32),
            pltpu.VMEM((CH,), jnp.int32),
            pltpu.VMEM((CH,), jnp.int32),
            pltpu.VMEM((3, CH), f32),
            pltpu.VMEM((CH, D), f32),
            pltpu.VMEM((CH, D), f32),
            pltpu.VMEM((CH,), f32),
            pltpu.VMEM((CH,), f32),
            pltpu.VMEM((CH,), f32),
            pltpu.SemaphoreType.DMA,
            pltpu.SemaphoreType.DMA,
        ],
    )
    hi, hj, rij, rji, cm2 = edge_sc(h, vflat, edge_index, evut)

    full = lambda shape: pl.BlockSpec(shape, lambda i: tuple(0 for _ in shape))
    msg, coeff = pl.pallas_call(
        _mlp_body,
        grid=(NBE,),
        in_specs=[
            pl.BlockSpec((BE, D), lambda i: (i, 0)),
            pl.BlockSpec((BE, D), lambda i: (i, 0)),
            pl.BlockSpec((BE, D), lambda i: (i, 0)),
            pl.BlockSpec((BE, 1), lambda i: (i, 0)),
            pl.BlockSpec((BE, 1), lambda i: (i, 0)),
            pl.BlockSpec((BE, 1), lambda i: (i, 0)),
            full((1, D)),
            full((1, D)),
            full((D, D)),
            full((1, D)),
            full((D, 2 * D)),
            full((D, 2 * D)),
            full((D, 2 * D)),
            full((1, 2 * D)),
            full((2 * D, D)),
            full((1, D)),
            full((D, 1)),
            full((1, 1)),
        ],
        out_specs=[
            pl.BlockSpec((BE, D), lambda i: (i, 0)),
            pl.BlockSpec((BE, 1), lambda i: (i, 0)),
        ],
        out_shape=[
            jax.ShapeDtypeStruct((E, D), f32),
            jax.ShapeDtypeStruct((E, 1), f32),
        ],
    )(hi, hj, edge_attr, rij.reshape(E, 1), rji.reshape(E, 1),
      cm2.reshape(E, 1), wng, weg, egW1d, eg_b1.reshape(1, D),
      ci, cj, ce, zb, mg_W2, mg_b2.reshape(1, D),
      pe_W, pe_b.reshape(1, 1))

    scatter_sc = pl.kernel(
        _scatter_sc_body,
        compiler_params=sc_params,
        out_type=[
            jax.ShapeDtypeStruct((NC, N, D), f32),
            jax.ShapeDtypeStruct((NC, 3 * N), f32),
        ],
        mesh=mesh,
        scratch_types=[
            pltpu.VMEM_SHARED((N, D), f32),
            pltpu.VMEM_SHARED((3 * N,), f32),
            pltpu.VMEM((CH, D), f32),
            pltpu.VMEM((CH,), jnp.int32),
            pltpu.VMEM((CH,), f32),
            pltpu.VMEM((3, CH), f32),
            pltpu.VMEM((3, CH), f32),
            pltpu.VMEM((3, CH), jnp.int32),
        ],
    )
    spart, vpart = scatter_sc(msg, coeff.reshape(E), evut, row,
                              zeros_s, zeros_v)

    snew, vnewt = pl.pallas_call(
        _final_body,
        out_shape=[
            jax.ShapeDtypeStruct((N, D), f32),
            jax.ShapeDtypeStruct((3, N), f32),
        ],
    )(s, spart, vt, vpart.reshape(NC, 3, N), up_W, up_b.reshape(1, D),
      ln_g.reshape(1, D), ln_b.reshape(1, D))

    return (snew, vnewt.T.reshape(N, 3))
